# baseline (speedup 1.0000x reference)
import functools

import jax
import jax.numpy as jnp
from jax import lax
from jax.experimental import pallas as pl
from jax.experimental.pallas import tpu as pltpu
from jax.experimental.pallas import tpu_sc as plsc

_NUM_BITS = 16
_N = 262144
_NUM_CORES = 2
_NUM_SUBCORES = 16
_NUM_WORKERS = _NUM_CORES * _NUM_SUBCORES
_ROWS_PER_WORKER = _N // _NUM_WORKERS
_CHUNKN = 2048
_NCH = _ROWS_PER_WORKER // _CHUNKN
_C128 = _CHUNKN // 128


def _compute_half(d_buf, stg, tr):
    @plsc.parallel_loop(0, _C128, unroll=4)
    def _(cc):
        for lb in range(8):
            dv = d_buf[pl.ds(cc * 128 + lb * 16, 16)]
            for r in range(8):
                bits = (dv >> (8 * tr + r)) & 1
                stg[r, pl.ds(cc * 128 + lb * 16, 16)] = bits.astype(jnp.float32)


def _body(d_hbm, out_hbm, in0, in1, st0, st1, si0, si1, so0, so1):
    wid = lax.axis_index("s") * _NUM_CORES + lax.axis_index("c")
    base = wid * _ROWS_PER_WORKER
    in_bufs = (in0, in1)
    stg_bufs = (st0, st1)
    in_sems = (si0, si1)
    out_sems = (so0, so1)

    in_copies = [None] * _NCH
    for c in range(min(2, _NCH)):
        in_copies[c] = pltpu.async_copy(
            d_hbm.at[pl.ds(base + c * _CHUNKN, _CHUNKN)], in_bufs[c], in_sems[c]
        )
    out_copies = [None] * (2 * _NCH)
    for c in range(_NCH):
        ib = c & 1
        in_copies[c].wait()
        for tr in range(2):
            s = c * 2 + tr
            sb = s & 1
            if s >= 2:
                out_copies[s - 2].wait()
            _compute_half(in_bufs[ib], stg_bufs[sb], tr)
            out_copies[s] = pltpu.async_copy(
                stg_bufs[sb],
                out_hbm.at[pl.ds(8 * tr, 8), pl.ds(base + c * _CHUNKN, _CHUNKN)],
                out_sems[sb],
            )
        if c + 2 < _NCH:
            in_copies[c + 2] = pltpu.async_copy(
                d_hbm.at[pl.ds(base + (c + 2) * _CHUNKN, _CHUNKN)],
                in_bufs[ib],
                in_sems[ib],
            )
    out_copies[2 * _NCH - 2].wait()
    out_copies[2 * _NCH - 1].wait()


@jax.jit
def kernel(distance):
    mesh = plsc.VectorSubcoreMesh(core_axis_name="c", subcore_axis_name="s")
    run = functools.partial(
        pl.kernel,
        out_type=jax.ShapeDtypeStruct((_NUM_BITS, _N), jnp.float32),
        mesh=mesh,
        compiler_params=pltpu.CompilerParams(
            needs_layout_passes=False, use_tc_tiling_on_sc=True
        ),
        scratch_types=[
            pltpu.VMEM((_CHUNKN,), jnp.int32),
            pltpu.VMEM((_CHUNKN,), jnp.int32),
            pltpu.VMEM((8, _CHUNKN), jnp.float32),
            pltpu.VMEM((8, _CHUNKN), jnp.float32),
            pltpu.SemaphoreType.DMA,
            pltpu.SemaphoreType.DMA,
            pltpu.SemaphoreType.DMA,
            pltpu.SemaphoreType.DMA,
        ],
    )(_body)
    return run(distance).T


# speedup vs baseline: 1.0906x; 1.0906x over previous
import functools

import jax
import jax.numpy as jnp
from jax import lax
from jax.experimental import pallas as pl
from jax.experimental.pallas import tpu as pltpu
from jax.experimental.pallas import tpu_sc as plsc

_NUM_BITS = 16
_N = 262144
_NUM_CORES = 2
_NUM_SUBCORES = 16
_NUM_WORKERS = _NUM_CORES * _NUM_SUBCORES
_ROWS_PER_WORKER = _N // _NUM_WORKERS
_CHUNKN = 4096
_NCH = _ROWS_PER_WORKER // _CHUNKN
_C128 = _CHUNKN // 128


def _compute_half(d_buf, stg, tr):
    @plsc.parallel_loop(0, _C128, unroll=4)
    def _(cc):
        for lb in range(8):
            dv = d_buf[pl.ds(cc * 128 + lb * 16, 16)]
            for r in range(8):
                bits = (dv >> (8 * tr + r)) & 1
                stg[r, pl.ds(cc * 128 + lb * 16, 16)] = bits.astype(jnp.float32)


def _body(d_hbm, out_hbm, in0, in1, st0, st1, si0, si1, so0, so1):
    wid = lax.axis_index("s") * _NUM_CORES + lax.axis_index("c")
    base = wid * _ROWS_PER_WORKER
    in_bufs = (in0, in1)
    stg_bufs = (st0, st1)
    in_sems = (si0, si1)
    out_sems = (so0, so1)

    in_copies = [
        pltpu.async_copy(
            d_hbm.at[pl.ds(base + c * _CHUNKN, _CHUNKN)], in_bufs[c], in_sems[c]
        )
        for c in range(_NCH)
    ]
    out_copies = [None] * (2 * _NCH)
    for c in range(_NCH):
        in_copies[c].wait()
        for tr in range(2):
            s = c * 2 + tr
            sb = s & 1
            if s >= 2:
                out_copies[s - 2].wait()
            _compute_half(in_bufs[c], stg_bufs[sb], tr)
            out_copies[s] = pltpu.async_copy(
                stg_bufs[sb],
                out_hbm.at[pl.ds(8 * tr, 8), pl.ds(base + c * _CHUNKN, _CHUNKN)],
                out_sems[sb],
            )
    out_copies[2 * _NCH - 2].wait()
    out_copies[2 * _NCH - 1].wait()


@jax.jit
def kernel(distance):
    mesh = plsc.VectorSubcoreMesh(core_axis_name="c", subcore_axis_name="s")
    run = functools.partial(
        pl.kernel,
        out_type=jax.ShapeDtypeStruct((_NUM_BITS, _N), jnp.float32),
        mesh=mesh,
        compiler_params=pltpu.CompilerParams(
            needs_layout_passes=False, use_tc_tiling_on_sc=True
        ),
        scratch_types=[
            pltpu.VMEM((_CHUNKN,), jnp.int32),
            pltpu.VMEM((_CHUNKN,), jnp.int32),
            pltpu.VMEM((8, _CHUNKN), jnp.float32),
            pltpu.VMEM((8, _CHUNKN), jnp.float32),
            pltpu.SemaphoreType.DMA,
            pltpu.SemaphoreType.DMA,
            pltpu.SemaphoreType.DMA,
            pltpu.SemaphoreType.DMA,
        ],
    )(_body)
    return run(distance).T
